# initial kernel scaffold (unmeasured)
import jax
import jax.numpy as jnp
from jax import lax
from jax.experimental import pallas as pl
from jax.experimental.pallas import tpu as pltpu

N_DEV = 32


def kernel(A, B):
    m_per, k = A.shape
    _, n = B.shape

    def body(a_ref, b_ref, out_ref, comm_ref, result_ref,
             send_sems, recv_sems, credit_sem, copy_sems):
        my = lax.axis_index("i")
        left = lax.rem(my - 1 + N_DEV, N_DEV)
        right = lax.rem(my + 1, N_DEV)

        barrier = pltpu.get_barrier_semaphore()
        for nbr in (left, right):
            pl.semaphore_signal(barrier, inc=1, device_id=(nbr,),
                                device_id_type=pl.DeviceIdType.MESH)
        pl.semaphore_wait(barrier, 2)

        copies = {}

        def compute_chunk(c, chunk, origin):
            slot = c % 2
            if c >= 2:
                copies[c - 2].wait()
            result_ref[slot] = jnp.dot(chunk, b_ref[...],
                                       preferred_element_type=jnp.float32)
            cp = pltpu.make_async_copy(
                result_ref.at[slot],
                out_ref.at[pl.ds(origin * m_per, m_per), :],
                copy_sems.at[slot],
            )
            cp.start()
            copies[c] = cp

        comm_ref[0] = a_ref[...]
        compute_chunk(0, a_ref[...], my)

        for h in range(N_DEV - 1):
            send_slot = h % 2
            recv_slot = (h + 1) % 2
            if h >= 1:
                pl.semaphore_wait(credit_sem, 1)
            rdma = pltpu.make_async_remote_copy(
                src_ref=comm_ref.at[send_slot],
                dst_ref=comm_ref.at[recv_slot],
                send_sem=send_sems.at[send_slot],
                recv_sem=recv_sems.at[recv_slot],
                device_id=(right,),
                device_id_type=pl.DeviceIdType.MESH,
            )
            rdma.start()
            rdma.wait()
            if h < N_DEV - 2:
                pl.semaphore_signal(credit_sem, inc=1, device_id=(left,),
                                    device_id_type=pl.DeviceIdType.MESH)
            origin = lax.rem(my - (h + 1) + N_DEV, N_DEV)
            compute_chunk(h + 1, comm_ref[recv_slot], origin)

        copies[N_DEV - 2].wait()
        copies[N_DEV - 1].wait()

    return pl.pallas_call(
        body,
        out_shape=jax.ShapeDtypeStruct((N_DEV * m_per, n), jnp.float32),
        in_specs=[
            pl.BlockSpec(memory_space=pltpu.VMEM),
            pl.BlockSpec(memory_space=pltpu.VMEM),
        ],
        out_specs=pl.BlockSpec(memory_space=pltpu.ANY),
        scratch_shapes=[
            pltpu.VMEM((2, m_per, k), jnp.float32),
            pltpu.VMEM((2, m_per, n), jnp.float32),
            pltpu.SemaphoreType.DMA((2,)),
            pltpu.SemaphoreType.DMA((2,)),
            pltpu.SemaphoreType.REGULAR,
            pltpu.SemaphoreType.DMA((2,)),
        ],
        compiler_params=pltpu.CompilerParams(collective_id=0),
    )(A, B)


# baseline (device time: 1011878 ns/iter reference)
import jax
import jax.numpy as jnp
from jax import lax
from jax.experimental import pallas as pl
from jax.experimental.pallas import tpu as pltpu

N_DEV = 32


def kernel(A, B):
    m_per, k = A.shape
    _, n = B.shape

    def body(a_ref, b_ref, out_ref, comm_ref, result_ref,
             send_sems, recv_sems, credit_sem, copy_sems):
        my = lax.axis_index("i")
        left = lax.rem(my - 1 + N_DEV, N_DEV)
        right = lax.rem(my + 1, N_DEV)

        barrier = pltpu.get_barrier_semaphore()
        for nbr in (left, right):
            pl.semaphore_signal(barrier, inc=1, device_id=(nbr,),
                                device_id_type=pl.DeviceIdType.MESH)
        pl.semaphore_wait(barrier, 2)

        copies = {}

        def compute_chunk(c, chunk, origin):
            slot = c % 2
            if c >= 2:
                copies[c - 2].wait()
            result_ref[slot] = jnp.dot(chunk, b_ref[...],
                                       preferred_element_type=jnp.float32)
            cp = pltpu.make_async_copy(
                result_ref.at[slot],
                out_ref.at[pl.ds(origin * m_per, m_per), :],
                copy_sems.at[slot],
            )
            cp.start()
            copies[c] = cp

        comm_ref[0] = a_ref[...]
        compute_chunk(0, a_ref[...], my)

        for h in range(N_DEV - 1):
            send_slot = h % 2
            recv_slot = (h + 1) % 2
            if h >= 1:
                pl.semaphore_wait(credit_sem, 1)
            rdma = pltpu.make_async_remote_copy(
                src_ref=comm_ref.at[send_slot],
                dst_ref=comm_ref.at[recv_slot],
                send_sem=send_sems.at[send_slot],
                recv_sem=recv_sems.at[recv_slot],
                device_id=(right,),
                device_id_type=pl.DeviceIdType.MESH,
            )
            rdma.start()
            rdma.wait()
            if h < N_DEV - 2:
                pl.semaphore_signal(credit_sem, inc=1, device_id=(left,),
                                    device_id_type=pl.DeviceIdType.MESH)
            origin = lax.rem(my - (h + 1) + N_DEV, N_DEV)
            compute_chunk(h + 1, comm_ref[recv_slot], origin)

        copies[N_DEV - 2].wait()
        copies[N_DEV - 1].wait()

    return pl.pallas_call(
        body,
        out_shape=jax.ShapeDtypeStruct((N_DEV * m_per, n), jnp.float32),
        in_specs=[
            pl.BlockSpec(memory_space=pltpu.VMEM),
            pl.BlockSpec(memory_space=pltpu.VMEM),
        ],
        out_specs=pl.BlockSpec(memory_space=pl.ANY),
        scratch_shapes=[
            pltpu.VMEM((2, m_per, k), jnp.float32),
            pltpu.VMEM((2, m_per, n), jnp.float32),
            pltpu.SemaphoreType.DMA((2,)),
            pltpu.SemaphoreType.DMA((2,)),
            pltpu.SemaphoreType.REGULAR,
            pltpu.SemaphoreType.DMA((2,)),
        ],
        compiler_params=pltpu.CompilerParams(collective_id=0),
    )(A, B)


# device time: 871275 ns/iter; 1.1614x vs baseline; 1.1614x over previous
import jax
import jax.numpy as jnp
from jax import lax
from jax.experimental import pallas as pl
from jax.experimental.pallas import tpu as pltpu

N_DEV = 32
S_F = 16
S_B = 15
NSLOT = 4


def kernel(A, B):
    m_per, k = A.shape
    _, n = B.shape

    def body(a_ref, b_ref, out_ref, comm_f, comm_b, result_ref,
             send_f, recv_f, send_b, recv_b, credit_f, credit_b, copy_sems):
        my = lax.axis_index("i")
        left = lax.rem(my - 1 + N_DEV, N_DEV)
        right = lax.rem(my + 1, N_DEV)

        barrier = pltpu.get_barrier_semaphore()
        for nbr in (left, right):
            pl.semaphore_signal(barrier, inc=1, device_id=(nbr,),
                                device_id_type=pl.DeviceIdType.MESH)
        pl.semaphore_wait(barrier, 2)

        copies = {}
        counter = [0]

        def compute_chunk(chunk, origin):
            c = counter[0]
            counter[0] += 1
            slot = c % NSLOT
            if c >= NSLOT:
                copies[c - NSLOT].wait()
            result_ref[slot] = jnp.dot(chunk, b_ref[...],
                                       preferred_element_type=jnp.float32)
            cp = pltpu.make_async_copy(
                result_ref.at[slot],
                out_ref.at[pl.ds(origin * m_per, m_per), :],
                copy_sems.at[slot],
            )
            cp.start()
            copies[c] = cp

        def make_rdma(direction, s):
            if direction == "f":
                comm, ssem, rsem, dst = comm_f, send_f, recv_f, right
            else:
                comm, ssem, rsem, dst = comm_b, send_b, recv_b, left
            src = a_ref if s == 1 else comm.at[(s - 1) % 2]
            return pltpu.make_async_remote_copy(
                src_ref=src,
                dst_ref=comm.at[s % 2],
                send_sem=ssem.at[s % 2],
                recv_sem=rsem.at[s % 2],
                device_id=(dst,),
                device_id_type=pl.DeviceIdType.MESH,
            )

        for s in range(1, S_F + 1):
            if s >= 3:
                pl.semaphore_wait(credit_f, 1)
                if s <= S_B:
                    pl.semaphore_wait(credit_b, 1)
            rf = make_rdma("f", s)
            rf.start()
            rb = None
            if s <= S_B:
                rb = make_rdma("b", s)
                rb.start()
            if s == 1:
                compute_chunk(a_ref[...], my)
            else:
                compute_chunk(comm_f[(s - 1) % 2],
                              lax.rem(my - (s - 1) + N_DEV, N_DEV))
                if s - 1 <= S_B:
                    compute_chunk(comm_b[(s - 1) % 2],
                                  lax.rem(my + (s - 1), N_DEV))
            rf.wait()
            if 2 <= s <= S_F - 1:
                pl.semaphore_signal(credit_f, inc=1, device_id=(left,),
                                    device_id_type=pl.DeviceIdType.MESH)
            if rb is not None:
                rb.wait()
                if 2 <= s <= S_B - 1:
                    pl.semaphore_signal(credit_b, inc=1, device_id=(right,),
                                        device_id_type=pl.DeviceIdType.MESH)

        compute_chunk(comm_f[S_F % 2], lax.rem(my - S_F + N_DEV, N_DEV))
        compute_chunk(comm_b[S_B % 2], lax.rem(my + S_B, N_DEV))

        for c in range(max(0, counter[0] - NSLOT), counter[0]):
            copies[c].wait()

    return pl.pallas_call(
        body,
        out_shape=jax.ShapeDtypeStruct((N_DEV * m_per, n), jnp.float32),
        in_specs=[
            pl.BlockSpec(memory_space=pltpu.VMEM),
            pl.BlockSpec(memory_space=pltpu.VMEM),
        ],
        out_specs=pl.BlockSpec(memory_space=pl.ANY),
        scratch_shapes=[
            pltpu.VMEM((2, m_per, k), jnp.float32),
            pltpu.VMEM((2, m_per, k), jnp.float32),
            pltpu.VMEM((NSLOT, m_per, n), jnp.float32),
            pltpu.SemaphoreType.DMA((2,)),
            pltpu.SemaphoreType.DMA((2,)),
            pltpu.SemaphoreType.DMA((2,)),
            pltpu.SemaphoreType.DMA((2,)),
            pltpu.SemaphoreType.REGULAR,
            pltpu.SemaphoreType.REGULAR,
            pltpu.SemaphoreType.DMA((NSLOT,)),
        ],
        compiler_params=pltpu.CompilerParams(collective_id=0),
    )(A, B)


# device time: 499480 ns/iter; 2.0259x vs baseline; 1.7444x over previous
import jax
import jax.numpy as jnp
from jax import lax
from jax.experimental import pallas as pl
from jax.experimental.pallas import tpu as pltpu

N_DEV = 32
S_F = 16
S_B = 15
NSLOT = 4


_PLANE_SNAKE = [(0, 0), (1, 0), (1, 1), (0, 1), (0, 2), (1, 2), (1, 3), (0, 3)]
_LOGICAL_OF_COORDS = {}
for _z in range(4):
    for _idx, (_x, _y) in enumerate(_PLANE_SNAKE):
        _LOGICAL_OF_COORDS[(_x, _y, _z)] = _z * 8 + _idx

_C8 = [(0, 0), (1, 0), (1, 1), (1, 2), (1, 3), (0, 3), (0, 2), (0, 1)]
_HAM_COORDS = []
for _p, (_x, _y) in enumerate(_C8):
    _zs = range(4) if _p % 2 == 0 else range(3, -1, -1)
    for _z in _zs:
        _HAM_COORDS.append((_x, _y, _z))
for _a, _b in zip(_HAM_COORDS, _HAM_COORDS[1:] + _HAM_COORDS[:1]):
    assert sum(abs(i - j) for i, j in zip(_a, _b)) == 1, (_a, _b)

HAM = [_LOGICAL_OF_COORDS[c] for c in _HAM_COORDS]
assert sorted(HAM) == list(range(N_DEV))


def kernel(A, B):
    m_per, k = A.shape
    _, n = B.shape

    def body(a_ref, b_ref, ham_ref, out_ref, comm_f, comm_b, result_ref,
             send_f, recv_f, send_b, recv_b, credit_f, credit_b, copy_sems):
        my = lax.axis_index("i")

        iota = lax.broadcasted_iota(jnp.int32, (1, N_DEV), 1)
        ham_arr = ham_ref[...]

        def ham_at(ring_pos):
            return jnp.sum(jnp.where(iota == ring_pos, ham_arr, 0))

        pos = jnp.sum(jnp.where(ham_arr == my, iota, 0))
        left = ham_at(lax.rem(pos - 1 + N_DEV, N_DEV))
        right = ham_at(lax.rem(pos + 1, N_DEV))

        barrier = pltpu.get_barrier_semaphore()
        for nbr in (left, right):
            pl.semaphore_signal(barrier, inc=1, device_id=(nbr,),
                                device_id_type=pl.DeviceIdType.MESH)
        pl.semaphore_wait(barrier, 2)

        copies = {}
        counter = [0]

        def compute_chunk(chunk, origin):
            c = counter[0]
            counter[0] += 1
            slot = c % NSLOT
            if c >= NSLOT:
                copies[c - NSLOT].wait()
            result_ref[slot] = jnp.dot(chunk, b_ref[...],
                                       preferred_element_type=jnp.float32)
            cp = pltpu.make_async_copy(
                result_ref.at[slot],
                out_ref.at[pl.ds(origin * m_per, m_per), :],
                copy_sems.at[slot],
            )
            cp.start()
            copies[c] = cp

        def make_rdma(direction, s):
            if direction == "f":
                comm, ssem, rsem, dst = comm_f, send_f, recv_f, right
            else:
                comm, ssem, rsem, dst = comm_b, send_b, recv_b, left
            src = a_ref if s == 1 else comm.at[(s - 1) % 2]
            return pltpu.make_async_remote_copy(
                src_ref=src,
                dst_ref=comm.at[s % 2],
                send_sem=ssem.at[s % 2],
                recv_sem=rsem.at[s % 2],
                device_id=(dst,),
                device_id_type=pl.DeviceIdType.MESH,
            )

        for s in range(1, S_F + 1):
            if s >= 3:
                pl.semaphore_wait(credit_f, 1)
                if s <= S_B:
                    pl.semaphore_wait(credit_b, 1)
            rf = make_rdma("f", s)
            rf.start()
            rb = None
            if s <= S_B:
                rb = make_rdma("b", s)
                rb.start()
            if s == 1:
                compute_chunk(a_ref[...], my)
            else:
                compute_chunk(comm_f[(s - 1) % 2],
                              ham_at(lax.rem(pos - (s - 1) + N_DEV, N_DEV)))
                if s - 1 <= S_B:
                    compute_chunk(comm_b[(s - 1) % 2],
                                  ham_at(lax.rem(pos + (s - 1), N_DEV)))
            rf.wait()
            if 2 <= s <= S_F - 1:
                pl.semaphore_signal(credit_f, inc=1, device_id=(left,),
                                    device_id_type=pl.DeviceIdType.MESH)
            if rb is not None:
                rb.wait()
                if 2 <= s <= S_B - 1:
                    pl.semaphore_signal(credit_b, inc=1, device_id=(right,),
                                        device_id_type=pl.DeviceIdType.MESH)

        compute_chunk(comm_f[S_F % 2], ham_at(lax.rem(pos - S_F + N_DEV, N_DEV)))
        compute_chunk(comm_b[S_B % 2], ham_at(lax.rem(pos + S_B, N_DEV)))

        for c in range(max(0, counter[0] - NSLOT), counter[0]):
            copies[c].wait()

    return pl.pallas_call(
        body,
        out_shape=jax.ShapeDtypeStruct((N_DEV * m_per, n), jnp.float32),
        in_specs=[
            pl.BlockSpec(memory_space=pltpu.VMEM),
            pl.BlockSpec(memory_space=pltpu.VMEM),
            pl.BlockSpec(memory_space=pltpu.VMEM),
        ],
        out_specs=pl.BlockSpec(memory_space=pl.ANY),
        scratch_shapes=[
            pltpu.VMEM((2, m_per, k), jnp.float32),
            pltpu.VMEM((2, m_per, k), jnp.float32),
            pltpu.VMEM((NSLOT, m_per, n), jnp.float32),
            pltpu.SemaphoreType.DMA((2,)),
            pltpu.SemaphoreType.DMA((2,)),
            pltpu.SemaphoreType.DMA((2,)),
            pltpu.SemaphoreType.DMA((2,)),
            pltpu.SemaphoreType.REGULAR,
            pltpu.SemaphoreType.REGULAR,
            pltpu.SemaphoreType.DMA((NSLOT,)),
        ],
        compiler_params=pltpu.CompilerParams(collective_id=0),
    )(A, B, jnp.array(HAM, dtype=jnp.int32).reshape(1, N_DEV))


# device time: 487642 ns/iter; 2.0750x vs baseline; 1.0243x over previous
import jax
import jax.numpy as jnp
from jax import lax
from jax.experimental import pallas as pl
from jax.experimental.pallas import tpu as pltpu

N_DEV = 32
S_F = 16
S_B = 15
NSLOT = 4
CSLOT = 4


_PLANE_SNAKE = [(0, 0), (1, 0), (1, 1), (0, 1), (0, 2), (1, 2), (1, 3), (0, 3)]
_LOGICAL_OF_COORDS = {}
for _z in range(4):
    for _idx, (_x, _y) in enumerate(_PLANE_SNAKE):
        _LOGICAL_OF_COORDS[(_x, _y, _z)] = _z * 8 + _idx

_C8 = [(0, 0), (1, 0), (1, 1), (1, 2), (1, 3), (0, 3), (0, 2), (0, 1)]
_HAM_COORDS = []
for _p, (_x, _y) in enumerate(_C8):
    _zs = range(4) if _p % 2 == 0 else range(3, -1, -1)
    for _z in _zs:
        _HAM_COORDS.append((_x, _y, _z))
for _a, _b in zip(_HAM_COORDS, _HAM_COORDS[1:] + _HAM_COORDS[:1]):
    assert sum(abs(i - j) for i, j in zip(_a, _b)) == 1, (_a, _b)

HAM = [_LOGICAL_OF_COORDS[c] for c in _HAM_COORDS]
assert sorted(HAM) == list(range(N_DEV))


def kernel(A, B):
    m_per, k = A.shape
    _, n = B.shape

    def body(a_ref, b_ref, ham_ref, out_ref, comm_f, comm_b, result_ref,
             send_f, recv_f, send_b, recv_b, credit_f, credit_b, copy_sems):
        my = lax.axis_index("i")

        iota = lax.broadcasted_iota(jnp.int32, (1, N_DEV), 1)
        ham_arr = ham_ref[...]

        def ham_at(ring_pos):
            return jnp.sum(jnp.where(iota == ring_pos, ham_arr, 0))

        pos = jnp.sum(jnp.where(ham_arr == my, iota, 0))
        left = ham_at(lax.rem(pos - 1 + N_DEV, N_DEV))
        right = ham_at(lax.rem(pos + 1, N_DEV))

        barrier = pltpu.get_barrier_semaphore()
        for nbr in (left, right):
            pl.semaphore_signal(barrier, inc=1, device_id=(nbr,),
                                device_id_type=pl.DeviceIdType.MESH)
        pl.semaphore_wait(barrier, 2)

        copies = {}
        counter = [0]

        def compute_chunk(chunk, origin):
            c = counter[0]
            counter[0] += 1
            slot = c % NSLOT
            if c >= NSLOT:
                copies[c - NSLOT].wait()
            result_ref[slot] = jnp.dot(chunk, b_ref[...],
                                       preferred_element_type=jnp.float32)
            cp = pltpu.make_async_copy(
                result_ref.at[slot],
                out_ref.at[pl.ds(origin * m_per, m_per), :],
                copy_sems.at[slot],
            )
            cp.start()
            copies[c] = cp

        def make_rdma(direction, s):
            if direction == "f":
                comm, ssem, rsem, dst = comm_f, send_f, recv_f, right
            else:
                comm, ssem, rsem, dst = comm_b, send_b, recv_b, left
            src = a_ref if s == 1 else comm.at[(s - 1) % CSLOT]
            return pltpu.make_async_remote_copy(
                src_ref=src,
                dst_ref=comm.at[s % CSLOT],
                send_sem=ssem.at[s % CSLOT],
                recv_sem=rsem.at[s % CSLOT],
                device_id=(dst,),
                device_id_type=pl.DeviceIdType.MESH,
            )

        rf = {}
        rb = {}
        for s in range(1, S_F + 1):
            if s >= 2:
                rf[s - 1].wait_recv()
            if s >= CSLOT + 1:
                pl.semaphore_wait(credit_f, 1)
            rf[s] = make_rdma("f", s)
            rf[s].start()
            if s >= 2:
                rf[s - 1].wait_send()
                if 3 <= s <= S_F - 2:
                    pl.semaphore_signal(credit_f, inc=1, device_id=(left,),
                                        device_id_type=pl.DeviceIdType.MESH)
            if s <= S_B:
                if s >= 2:
                    rb[s - 1].wait_recv()
                if s >= CSLOT + 1:
                    pl.semaphore_wait(credit_b, 1)
                rb[s] = make_rdma("b", s)
                rb[s].start()
                if s >= 2:
                    rb[s - 1].wait_send()
                    if 3 <= s <= S_B - 2:
                        pl.semaphore_signal(credit_b, inc=1,
                                            device_id=(right,),
                                            device_id_type=pl.DeviceIdType.MESH)
            if s == 1:
                compute_chunk(a_ref[...], my)
            else:
                compute_chunk(comm_f[(s - 1) % CSLOT],
                              ham_at(lax.rem(pos - (s - 1) + N_DEV, N_DEV)))
                if s - 1 <= S_B:
                    compute_chunk(comm_b[(s - 1) % CSLOT],
                                  ham_at(lax.rem(pos + (s - 1), N_DEV)))

        rf[S_F].wait_recv()
        rf[S_F].wait_send()
        rb[S_B].wait_recv()
        rb[S_B].wait_send()
        compute_chunk(comm_f[S_F % CSLOT], ham_at(lax.rem(pos - S_F + N_DEV, N_DEV)))
        compute_chunk(comm_b[S_B % CSLOT], ham_at(lax.rem(pos + S_B, N_DEV)))

        for c in range(max(0, counter[0] - NSLOT), counter[0]):
            copies[c].wait()

    return pl.pallas_call(
        body,
        out_shape=jax.ShapeDtypeStruct((N_DEV * m_per, n), jnp.float32),
        in_specs=[
            pl.BlockSpec(memory_space=pltpu.VMEM),
            pl.BlockSpec(memory_space=pltpu.VMEM),
            pl.BlockSpec(memory_space=pltpu.VMEM),
        ],
        out_specs=pl.BlockSpec(memory_space=pl.ANY),
        scratch_shapes=[
            pltpu.VMEM((CSLOT, m_per, k), jnp.float32),
            pltpu.VMEM((CSLOT, m_per, k), jnp.float32),
            pltpu.VMEM((NSLOT, m_per, n), jnp.float32),
            pltpu.SemaphoreType.DMA((CSLOT,)),
            pltpu.SemaphoreType.DMA((CSLOT,)),
            pltpu.SemaphoreType.DMA((CSLOT,)),
            pltpu.SemaphoreType.DMA((CSLOT,)),
            pltpu.SemaphoreType.REGULAR,
            pltpu.SemaphoreType.REGULAR,
            pltpu.SemaphoreType.DMA((NSLOT,)),
        ],
        compiler_params=pltpu.CompilerParams(
            collective_id=0, vmem_limit_bytes=50 * 2**20),
    )(A, B, jnp.array(HAM, dtype=jnp.int32).reshape(1, N_DEV))


# device time: 482536 ns/iter; 2.0970x vs baseline; 1.0106x over previous
import jax
import jax.numpy as jnp
from jax import lax
from jax.experimental import pallas as pl
from jax.experimental.pallas import tpu as pltpu

N_DEV = 32
S_F = 16
S_B = 15
NSLOT = 4
CSLOT = 4


_PLANE_SNAKE = [(0, 0), (1, 0), (1, 1), (0, 1), (0, 2), (1, 2), (1, 3), (0, 3)]
_LOGICAL_OF_COORDS = {}
for _z in range(4):
    for _idx, (_x, _y) in enumerate(_PLANE_SNAKE):
        _LOGICAL_OF_COORDS[(_x, _y, _z)] = _z * 8 + _idx

_C8 = [(0, 0), (1, 0), (1, 1), (1, 2), (1, 3), (0, 3), (0, 2), (0, 1)]
_HAM_COORDS = []
for _p, (_x, _y) in enumerate(_C8):
    _zs = range(4) if _p % 2 == 0 else range(3, -1, -1)
    for _z in _zs:
        _HAM_COORDS.append((_x, _y, _z))
for _a, _b in zip(_HAM_COORDS, _HAM_COORDS[1:] + _HAM_COORDS[:1]):
    assert sum(abs(i - j) for i, j in zip(_a, _b)) == 1, (_a, _b)

HAM = [_LOGICAL_OF_COORDS[c] for c in _HAM_COORDS]
assert sorted(HAM) == list(range(N_DEV))


def kernel(A, B):
    m_per, k = A.shape
    _, n = B.shape

    def body(a_ref, b_ref, ham_ref, out_ref, comm_f, comm_b, result_ref,
             send_f, recv_f, send_b, recv_b, credit_f, credit_b, copy_sems):
        my = lax.axis_index("i")

        iota = lax.broadcasted_iota(jnp.int32, (1, N_DEV), 1)
        ham_arr = ham_ref[...]

        def ham_at(ring_pos):
            return jnp.sum(jnp.where(iota == ring_pos, ham_arr, 0))

        pos = jnp.sum(jnp.where(ham_arr == my, iota, 0))
        left = ham_at(lax.rem(pos - 1 + N_DEV, N_DEV))
        right = ham_at(lax.rem(pos + 1, N_DEV))

        barrier = pltpu.get_barrier_semaphore()
        for nbr in (left, right):
            pl.semaphore_signal(barrier, inc=1, device_id=(nbr,),
                                device_id_type=pl.DeviceIdType.MESH)
        pl.semaphore_wait(barrier, 2)

        copies = {}
        counter = [0]

        import os
        _COMM_ONLY = os.environ.get("COMM_ONLY") == "1"

        def compute_chunk(chunk, origin):
            if _COMM_ONLY:
                return
            c = counter[0]
            counter[0] += 1
            slot = c % NSLOT
            if c >= NSLOT:
                copies[c - NSLOT].wait()
            result_ref[slot] = jnp.dot(chunk, b_ref[...],
                                       preferred_element_type=jnp.float32)
            cp = pltpu.make_async_copy(
                result_ref.at[slot],
                out_ref.at[pl.ds(origin * m_per, m_per), :],
                copy_sems.at[slot],
            )
            cp.start()
            copies[c] = cp

        def make_rdma(direction, s):
            if direction == "f":
                comm, ssem, rsem, dst = comm_f, send_f, recv_f, right
            else:
                comm, ssem, rsem, dst = comm_b, send_b, recv_b, left
            src = a_ref if s == 1 else comm.at[(s - 1) % CSLOT]
            return pltpu.make_async_remote_copy(
                src_ref=src,
                dst_ref=comm.at[s % CSLOT],
                send_sem=ssem.at[s % CSLOT],
                recv_sem=rsem.at[s % CSLOT],
                device_id=(dst,),
                device_id_type=pl.DeviceIdType.MESH,
            )

        rf = {}
        rb = {}
        for s in range(1, S_F + 1):
            if s >= 2:
                rf[s - 1].wait_recv()
            if s >= CSLOT + 1:
                pl.semaphore_wait(credit_f, 1)
            rf[s] = make_rdma("f", s)
            rf[s].start()
            if s >= 2:
                rf[s - 1].wait_send()
                if 3 <= s <= S_F - 2:
                    pl.semaphore_signal(credit_f, inc=1, device_id=(left,),
                                        device_id_type=pl.DeviceIdType.MESH)
            if s <= S_B:
                if s >= 2:
                    rb[s - 1].wait_recv()
                if s >= CSLOT + 1:
                    pl.semaphore_wait(credit_b, 1)
                rb[s] = make_rdma("b", s)
                rb[s].start()
                if s >= 2:
                    rb[s - 1].wait_send()
                    if 3 <= s <= S_B - 2:
                        pl.semaphore_signal(credit_b, inc=1,
                                            device_id=(right,),
                                            device_id_type=pl.DeviceIdType.MESH)
            if s == 1:
                compute_chunk(a_ref[...], my)
            else:
                compute_chunk(comm_f[(s - 1) % CSLOT],
                              ham_at(lax.rem(pos - (s - 1) + N_DEV, N_DEV)))
                if s - 1 <= S_B:
                    compute_chunk(comm_b[(s - 1) % CSLOT],
                                  ham_at(lax.rem(pos + (s - 1), N_DEV)))

        rf[S_F].wait_recv()
        rf[S_F].wait_send()
        rb[S_B].wait_recv()
        rb[S_B].wait_send()
        compute_chunk(comm_f[S_F % CSLOT], ham_at(lax.rem(pos - S_F + N_DEV, N_DEV)))
        compute_chunk(comm_b[S_B % CSLOT], ham_at(lax.rem(pos + S_B, N_DEV)))

        for c in range(max(0, counter[0] - NSLOT), counter[0]):
            copies[c].wait()

    return pl.pallas_call(
        body,
        out_shape=jax.ShapeDtypeStruct((N_DEV * m_per, n), jnp.float32),
        in_specs=[
            pl.BlockSpec(memory_space=pltpu.VMEM),
            pl.BlockSpec(memory_space=pltpu.VMEM),
            pl.BlockSpec(memory_space=pltpu.VMEM),
        ],
        out_specs=pl.BlockSpec(memory_space=pl.ANY),
        scratch_shapes=[
            pltpu.VMEM((CSLOT, m_per, k), jnp.float32),
            pltpu.VMEM((CSLOT, m_per, k), jnp.float32),
            pltpu.VMEM((NSLOT, m_per, n), jnp.float32),
            pltpu.SemaphoreType.DMA((CSLOT,)),
            pltpu.SemaphoreType.DMA((CSLOT,)),
            pltpu.SemaphoreType.DMA((CSLOT,)),
            pltpu.SemaphoreType.DMA((CSLOT,)),
            pltpu.SemaphoreType.REGULAR,
            pltpu.SemaphoreType.REGULAR,
            pltpu.SemaphoreType.DMA((NSLOT,)),
        ],
        compiler_params=pltpu.CompilerParams(
            collective_id=0, vmem_limit_bytes=50 * 2**20),
    )(A, B, jnp.array(HAM, dtype=jnp.int32).reshape(1, N_DEV))


# device time: 459456 ns/iter; 2.2023x vs baseline; 1.0502x over previous
import jax
import jax.numpy as jnp
from jax import lax
from jax.experimental import pallas as pl
from jax.experimental.pallas import tpu as pltpu

N_DEV = 32
S_F = 16
S_B = 15
NSLOT = 4
CSLOT = 4
SPLIT = 2


_PLANE_SNAKE = [(0, 0), (1, 0), (1, 1), (0, 1), (0, 2), (1, 2), (1, 3), (0, 3)]
_LOGICAL_OF_COORDS = {}
for _z in range(4):
    for _idx, (_x, _y) in enumerate(_PLANE_SNAKE):
        _LOGICAL_OF_COORDS[(_x, _y, _z)] = _z * 8 + _idx

_C8 = [(0, 0), (1, 0), (1, 1), (1, 2), (1, 3), (0, 3), (0, 2), (0, 1)]
_HAM_COORDS = []
for _p, (_x, _y) in enumerate(_C8):
    _zs = range(4) if _p % 2 == 0 else range(3, -1, -1)
    for _z in _zs:
        _HAM_COORDS.append((_x, _y, _z))
for _a, _b in zip(_HAM_COORDS, _HAM_COORDS[1:] + _HAM_COORDS[:1]):
    assert sum(abs(i - j) for i, j in zip(_a, _b)) == 1, (_a, _b)

HAM = [_LOGICAL_OF_COORDS[c] for c in _HAM_COORDS]
assert sorted(HAM) == list(range(N_DEV))


def kernel(A, B):
    m_per, k = A.shape
    _, n = B.shape

    def body(a_ref, b_ref, ham_ref, out_ref, comm_f, comm_b, result_ref,
             send_f, recv_f, send_b, recv_b, credit_f, credit_b, copy_sems):
        my = lax.axis_index("i")

        iota = lax.broadcasted_iota(jnp.int32, (1, N_DEV), 1)
        ham_arr = ham_ref[...]

        def ham_at(ring_pos):
            return jnp.sum(jnp.where(iota == ring_pos, ham_arr, 0))

        pos = jnp.sum(jnp.where(ham_arr == my, iota, 0))
        left = ham_at(lax.rem(pos - 1 + N_DEV, N_DEV))
        right = ham_at(lax.rem(pos + 1, N_DEV))

        barrier = pltpu.get_barrier_semaphore()
        for nbr in (left, right):
            pl.semaphore_signal(barrier, inc=1, device_id=(nbr,),
                                device_id_type=pl.DeviceIdType.MESH)
        pl.semaphore_wait(barrier, 2)

        copies = {}
        counter = [0]

        import os
        _COMM_ONLY = os.environ.get("COMM_ONLY") == "1"

        def compute_chunk(chunk, origin):
            if _COMM_ONLY:
                return
            c = counter[0]
            counter[0] += 1
            slot = c % NSLOT
            if c >= NSLOT:
                copies[c - NSLOT].wait()
            result_ref[slot] = jnp.dot(chunk, b_ref[...],
                                       preferred_element_type=jnp.float32)
            cp = pltpu.make_async_copy(
                result_ref.at[slot],
                out_ref.at[pl.ds(origin * m_per, m_per), :],
                copy_sems.at[slot],
            )
            cp.start()
            copies[c] = cp

        m_half = m_per // SPLIT

        def make_rdma(direction, s, h):
            if direction == "f":
                comm, ssem, rsem, dst = comm_f, send_f, recv_f, right
            else:
                comm, ssem, rsem, dst = comm_b, send_b, recv_b, left
            rows = pl.ds(h * m_half, m_half)
            src_buf = a_ref if s == 1 else comm.at[(s - 1) % CSLOT]
            return pltpu.make_async_remote_copy(
                src_ref=src_buf.at[rows, :],
                dst_ref=comm.at[s % CSLOT, rows, :],
                send_sem=ssem.at[s % CSLOT, h],
                recv_sem=rsem.at[s % CSLOT, h],
                device_id=(dst,),
                device_id_type=pl.DeviceIdType.MESH,
            )

        rf = {}
        rb = {}
        for s in range(1, S_F + 1):
            for h in range(SPLIT):
                if h == 0 and s >= CSLOT + 1:
                    pl.semaphore_wait(credit_f, 1)
                if s >= 2:
                    rf[s - 1, h].wait_recv()
                rf[s, h] = make_rdma("f", s, h)
                rf[s, h].start()
                if s <= S_B:
                    if h == 0 and s >= CSLOT + 1:
                        pl.semaphore_wait(credit_b, 1)
                    if s >= 2:
                        rb[s - 1, h].wait_recv()
                    rb[s, h] = make_rdma("b", s, h)
                    rb[s, h].start()
            if s >= 2:
                for h in range(SPLIT):
                    rf[s - 1, h].wait_send()
                if 3 <= s <= S_F - 2:
                    pl.semaphore_signal(credit_f, inc=1, device_id=(left,),
                                        device_id_type=pl.DeviceIdType.MESH)
                if s <= S_B:
                    for h in range(SPLIT):
                        rb[s - 1, h].wait_send()
                    if 3 <= s <= S_B - 2:
                        pl.semaphore_signal(credit_b, inc=1,
                                            device_id=(right,),
                                            device_id_type=pl.DeviceIdType.MESH)
            if s == 1:
                compute_chunk(a_ref[...], my)
            else:
                compute_chunk(comm_f[(s - 1) % CSLOT],
                              ham_at(lax.rem(pos - (s - 1) + N_DEV, N_DEV)))
                if s - 1 <= S_B:
                    compute_chunk(comm_b[(s - 1) % CSLOT],
                                  ham_at(lax.rem(pos + (s - 1), N_DEV)))

        for h in range(SPLIT):
            rf[S_F, h].wait_recv()
            rf[S_F, h].wait_send()
            rb[S_B, h].wait_recv()
            rb[S_B, h].wait_send()
        compute_chunk(comm_f[S_F % CSLOT], ham_at(lax.rem(pos - S_F + N_DEV, N_DEV)))
        compute_chunk(comm_b[S_B % CSLOT], ham_at(lax.rem(pos + S_B, N_DEV)))

        for c in range(max(0, counter[0] - NSLOT), counter[0]):
            copies[c].wait()

    return pl.pallas_call(
        body,
        out_shape=jax.ShapeDtypeStruct((N_DEV * m_per, n), jnp.float32),
        in_specs=[
            pl.BlockSpec(memory_space=pltpu.VMEM),
            pl.BlockSpec(memory_space=pltpu.VMEM),
            pl.BlockSpec(memory_space=pltpu.VMEM),
        ],
        out_specs=pl.BlockSpec(memory_space=pl.ANY),
        scratch_shapes=[
            pltpu.VMEM((CSLOT, m_per, k), jnp.float32),
            pltpu.VMEM((CSLOT, m_per, k), jnp.float32),
            pltpu.VMEM((NSLOT, m_per, n), jnp.float32),
            pltpu.SemaphoreType.DMA((CSLOT, SPLIT)),
            pltpu.SemaphoreType.DMA((CSLOT, SPLIT)),
            pltpu.SemaphoreType.DMA((CSLOT, SPLIT)),
            pltpu.SemaphoreType.DMA((CSLOT, SPLIT)),
            pltpu.SemaphoreType.REGULAR,
            pltpu.SemaphoreType.REGULAR,
            pltpu.SemaphoreType.DMA((NSLOT,)),
        ],
        compiler_params=pltpu.CompilerParams(
            collective_id=0, vmem_limit_bytes=50 * 2**20),
    )(A, B, jnp.array(HAM, dtype=jnp.int32).reshape(1, N_DEV))
